# baseline (speedup 1.0000x reference)
import functools

import jax
import jax.numpy as jnp
from jax import lax
from jax.experimental import pallas as pl
from jax.experimental.pallas import tpu as pltpu
from jax.experimental.pallas import tpu_sc as plsc


def _tc_argmax(x_flat, codebook, mt=256, kt=2048):
    m, d = x_flat.shape
    k_total = codebook.shape[0]
    n_mt, n_kt = m // mt, k_total // kt

    def body(x_ref, cb_ref, cn_ref, idx_ref, ms_ref, rmax, ridx):
        ki = pl.program_id(0)
        mi = pl.program_id(1)
        xb = x_ref[...]
        xn = xb / jnp.maximum(
            jnp.sqrt(jnp.sum(xb * xb, axis=1, keepdims=True)), 1e-12)
        cb = cb_ref[...]
        cn = cb / jnp.maximum(
            jnp.sqrt(jnp.sum(cb * cb, axis=1, keepdims=True)), 1e-12)
        cn_ref[...] = cn
        sim = lax.dot_general(xn, cn, (((1,), (1,)), ((), ())),
                              preferred_element_type=jnp.float32)
        tmax = jnp.max(sim, axis=1)
        targ = jnp.argmax(sim, axis=1).astype(jnp.int32) + ki * kt
        prev_max = jnp.where(ki == 0, -jnp.inf, rmax[mi])
        prev_idx = ridx[mi]
        better = tmax > prev_max
        new_max = jnp.where(better, tmax, prev_max)
        new_idx = jnp.where(better, targ, prev_idx)
        rmax[mi] = new_max
        ridx[mi] = new_idx
        idx_ref[...] = new_idx
        ms_ref[...] = new_max

    return pl.pallas_call(
        body,
        grid=(n_kt, n_mt),
        in_specs=[
            pl.BlockSpec((mt, d), lambda ki, mi: (mi, 0)),
            pl.BlockSpec((kt, d), lambda ki, mi: (ki, 0)),
        ],
        out_specs=[
            pl.BlockSpec((kt, d), lambda ki, mi: (ki, 0)),
            pl.BlockSpec((mt,), lambda ki, mi: (mi,)),
            pl.BlockSpec((mt,), lambda ki, mi: (mi,)),
        ],
        out_shape=[
            jax.ShapeDtypeStruct((k_total, d), jnp.float32),
            jax.ShapeDtypeStruct((m,), jnp.int32),
            jax.ShapeDtypeStruct((m,), jnp.float32),
        ],
        scratch_shapes=[
            pltpu.VMEM((n_mt, mt), jnp.float32),
            pltpu.VMEM((n_mt, mt), jnp.int32),
        ],
    )(x_flat, codebook)


def _sc_gather_hist(cn, idx):
    k_total, d = cn.shape
    m = idx.shape[0]
    info = plsc.get_sparse_core_info()
    nc, ns = info.num_cores, info.num_subcores
    nw = nc * ns
    rows_w = m // nw
    ch = 128
    n_ch = rows_w // ch
    hr, hc = k_total // 128, 128

    mesh = plsc.VectorSubcoreMesh(core_axis_name="c", subcore_axis_name="s")

    @functools.partial(
        pl.kernel,
        out_type=[
            jax.ShapeDtypeStruct((m, d), jnp.float32),
            jax.ShapeDtypeStruct((nc, hr, hc), jnp.float32),
        ],
        mesh=mesh,
        compiler_params=pltpu.CompilerParams(needs_layout_passes=False),
        scratch_types=[
            pltpu.VMEM((rows_w,), jnp.int32),
            pltpu.VMEM((ch, d), jnp.float32),
            pltpu.VMEM((ch, d), jnp.float32),
            pltpu.VMEM((hr, hc), jnp.float32),
            pltpu.VMEM((hr,), jnp.int32),
            pltpu.VMEM_SHARED((hr, hc), jnp.float32),
            pltpu.SemaphoreType.DMA,
            pltpu.SemaphoreType.DMA,
        ],
    )
    def sc_body(cn_hbm, idx_hbm, zq_hbm, cnt_hbm,
                idx_v, buf0, buf1, hist_v, rowid_v, shared_hist, sem0, sem1):
        ci = lax.axis_index("c")
        si = lax.axis_index("s")
        wid = si * nc + ci
        base = wid * rows_w

        pltpu.sync_copy(idx_hbm.at[pl.ds(base, rows_w)], idx_v)

        zeros16 = jnp.zeros((16,), jnp.float32)

        def zero_body(t, _):
            r = t // (hc // 16)
            c = (t % (hc // 16)) * 16
            hist_v[r, pl.ds(c, 16)] = zeros16
            return 0

        lax.fori_loop(0, hr * (hc // 16), zero_body, 0)
        for j in range(hr // 16):
            rowid_v[pl.ds(j * 16, 16)] = lax.iota(jnp.int32, 16) + j * 16

        @pl.when(si == 0)
        def _():
            pltpu.sync_copy(hist_v, shared_hist)

        bufs = (buf0, buf1)
        sems = (sem0, sem1)
        cp = pltpu.async_copy(cn_hbm.at[idx_v.at[pl.ds(0, ch)]], buf0, sem0)
        for c in range(n_ch):
            nxt = None
            if c + 1 < n_ch:
                nxt = pltpu.async_copy(
                    cn_hbm.at[idx_v.at[pl.ds((c + 1) * ch, ch)]],
                    bufs[(c + 1) % 2], sems[(c + 1) % 2])
            cp.wait()
            pltpu.sync_copy(bufs[c % 2], zq_hbm.at[pl.ds(base + c * ch, ch)])
            cp = nxt

        lane = lax.iota(jnp.int32, 16)
        ones16 = jnp.ones((16,), jnp.float32)

        def hist_body(v, _):
            vec = idx_v[pl.ds(v * 16, 16)]
            row = lax.shift_right_logical(vec, 7)
            col = lax.bitwise_and(vec, 127)
            for j in range(16):
                plsc.addupdate_scatter(hist_v, [row, col], ones16,
                                       mask=lane == j)
            return 0

        lax.fori_loop(0, rows_w // 16, hist_body, 0)

        plsc.subcore_barrier()
        pltpu.sync_copy(hist_v, shared_hist.at[rowid_v], add=True)
        plsc.subcore_barrier()

        @pl.when(si == 0)
        def _():
            pltpu.sync_copy(shared_hist, cnt_hbm.at[ci])

    return sc_body(cn, idx)


def _tc_scalars(ms2d, counts, m, d):
    def body(ms_ref, cnt_ref, loss_ref, perp_ref):
        mean_s = jnp.sum(ms_ref[...]) * (1.0 / m)
        loss = 1.25 * (2.0 - 2.0 * mean_s) * (1.0 / d)
        loss_ref[...] = jnp.broadcast_to(loss, (1, 1))
        cnt = cnt_ref[0] + cnt_ref[1]
        e = cnt * (1.0 / m)
        ent = -jnp.sum(e * jnp.log(e + 1e-10))
        perp_ref[...] = jnp.broadcast_to(jnp.exp(ent), (1, 1))

    return pl.pallas_call(
        body,
        out_shape=[
            jax.ShapeDtypeStruct((1, 1), jnp.float32),
            jax.ShapeDtypeStruct((1, 1), jnp.float32),
        ],
    )(ms2d, counts)


def kernel(x, codebook):
    b, n, d = x.shape
    m = b * n
    x_flat = x.reshape(m, d)
    cn, idx, maxsim = _tc_argmax(x_flat, codebook)
    zq, counts = _sc_gather_hist(cn, idx)
    loss, perp = _tc_scalars(maxsim.reshape(128, m // 128), counts, m, d)
    return zq.reshape(b, n, d), loss.reshape(()), perp.reshape(())


# speedup vs baseline: 2.3885x; 2.3885x over previous
import functools

import jax
import jax.numpy as jnp
from jax import lax
from jax.experimental import pallas as pl
from jax.experimental.pallas import tpu as pltpu
from jax.experimental.pallas import tpu_sc as plsc


def _tc_argmax(x_flat, codebook, mt=256):
    m, d = x_flat.shape
    k_total = codebook.shape[0]
    n_mt = m // mt

    def body(x_ref, cb_ref, cn_ref, idx_ref, ms_ref):
        mi = pl.program_id(0)

        @pl.when(mi == 0)
        def _():
            cb = cb_ref[...]
            cn_ref[...] = cb / jnp.maximum(
                jnp.sqrt(jnp.sum(cb * cb, axis=1, keepdims=True)), 1e-12)

        xb = x_ref[...]
        sim = lax.dot_general(xb, cn_ref[...], (((1,), (1,)), ((), ())),
                              preferred_element_type=jnp.float32)
        tmax = jnp.max(sim, axis=1)
        targ = jnp.argmax(sim, axis=1).astype(jnp.int32)
        nrm = jnp.sqrt(jnp.sum(xb * xb, axis=1))
        idx_ref[...] = targ
        ms_ref[...] = tmax / jnp.maximum(nrm, 1e-12)

    return pl.pallas_call(
        body,
        grid=(n_mt,),
        in_specs=[
            pl.BlockSpec((mt, d), lambda mi: (mi, 0)),
            pl.BlockSpec((k_total, d), lambda mi: (0, 0)),
        ],
        out_specs=[
            pl.BlockSpec((k_total, d), lambda mi: (0, 0)),
            pl.BlockSpec((mt,), lambda mi: (mi,)),
            pl.BlockSpec((mt,), lambda mi: (mi,)),
        ],
        out_shape=[
            jax.ShapeDtypeStruct((k_total, d), jnp.float32),
            jax.ShapeDtypeStruct((m,), jnp.int32),
            jax.ShapeDtypeStruct((m,), jnp.float32),
        ],
        compiler_params=pltpu.CompilerParams(
            vmem_limit_bytes=100 * 1024 * 1024),
    )(x_flat, codebook)


def _sc_gather_hist(cn, idx):
    k_total, d = cn.shape
    m = idx.shape[0]
    info = plsc.get_sparse_core_info()
    nc, ns = info.num_cores, info.num_subcores
    nw = nc * ns
    rows_w = m // nw
    ch = 128
    n_ch = rows_w // ch
    hr, hc = k_total // 128, 128

    mesh = plsc.VectorSubcoreMesh(core_axis_name="c", subcore_axis_name="s")

    @functools.partial(
        pl.kernel,
        out_type=[
            jax.ShapeDtypeStruct((m, d), jnp.float32),
            jax.ShapeDtypeStruct((nc, hr, hc), jnp.float32),
        ],
        mesh=mesh,
        compiler_params=pltpu.CompilerParams(needs_layout_passes=False),
        scratch_types=[
            pltpu.VMEM((rows_w,), jnp.int32),
            pltpu.VMEM((ch, d), jnp.float32),
            pltpu.VMEM((ch, d), jnp.float32),
            pltpu.VMEM((hr, hc), jnp.float32),
            pltpu.VMEM((hr,), jnp.int32),
            pltpu.VMEM_SHARED((hr, hc), jnp.float32),
            pltpu.SemaphoreType.DMA,
            pltpu.SemaphoreType.DMA,
        ],
    )
    def sc_body(cn_hbm, idx_hbm, zq_hbm, cnt_hbm,
                idx_v, buf0, buf1, hist_v, rowid_v, shared_hist, sem0, sem1):
        ci = lax.axis_index("c")
        si = lax.axis_index("s")
        wid = si * nc + ci
        base = wid * rows_w

        pltpu.sync_copy(idx_hbm.at[pl.ds(base, rows_w)], idx_v)

        zeros16 = jnp.zeros((16,), jnp.float32)

        def zero_body(t, _):
            r = t // (hc // 16)
            c = (t % (hc // 16)) * 16
            hist_v[r, pl.ds(c, 16)] = zeros16
            return 0

        lax.fori_loop(0, hr * (hc // 16), zero_body, 0)
        for j in range(hr // 16):
            rowid_v[pl.ds(j * 16, 16)] = lax.iota(jnp.int32, 16) + j * 16

        @pl.when(si == 0)
        def _():
            pltpu.sync_copy(hist_v, shared_hist)

        bufs = (buf0, buf1)
        sems = (sem0, sem1)
        cp = pltpu.async_copy(cn_hbm.at[idx_v.at[pl.ds(0, ch)]], buf0, sem0)
        for c in range(n_ch):
            nxt = None
            if c + 1 < n_ch:
                nxt = pltpu.async_copy(
                    cn_hbm.at[idx_v.at[pl.ds((c + 1) * ch, ch)]],
                    bufs[(c + 1) % 2], sems[(c + 1) % 2])
            cp.wait()
            pltpu.sync_copy(bufs[c % 2], zq_hbm.at[pl.ds(base + c * ch, ch)])
            cp = nxt

        lane = lax.iota(jnp.int32, 16)
        ones16 = jnp.ones((16,), jnp.float32)

        def hist_body(v, _):
            vec = idx_v[pl.ds(v * 16, 16)]
            row = lax.shift_right_logical(vec, 7)
            col = lax.bitwise_and(vec, 127)
            for j in range(16):
                plsc.addupdate_scatter(hist_v, [row, col], ones16,
                                       mask=lane == j)
            return 0

        lax.fori_loop(0, rows_w // 16, hist_body, 0)

        plsc.subcore_barrier()
        pltpu.sync_copy(hist_v, shared_hist.at[rowid_v], add=True)
        plsc.subcore_barrier()

        @pl.when(si == 0)
        def _():
            pltpu.sync_copy(shared_hist, cnt_hbm.at[ci])

    return sc_body(cn, idx)


def _tc_scalars(ms2d, counts, m, d):
    def body(ms_ref, cnt_ref, loss_ref, perp_ref):
        mean_s = jnp.sum(ms_ref[...]) * (1.0 / m)
        loss = 1.25 * (2.0 - 2.0 * mean_s) * (1.0 / d)
        loss_ref[...] = jnp.broadcast_to(loss, (1, 1))
        cnt = cnt_ref[0] + cnt_ref[1]
        e = cnt * (1.0 / m)
        ent = -jnp.sum(e * jnp.log(e + 1e-10))
        perp_ref[...] = jnp.broadcast_to(jnp.exp(ent), (1, 1))

    return pl.pallas_call(
        body,
        out_shape=[
            jax.ShapeDtypeStruct((1, 1), jnp.float32),
            jax.ShapeDtypeStruct((1, 1), jnp.float32),
        ],
    )(ms2d, counts)


def kernel(x, codebook):
    b, n, d = x.shape
    m = b * n
    x_flat = x.reshape(m, d)
    cn, idx, maxsim = _tc_argmax(x_flat, codebook)
    zq, counts = _sc_gather_hist(cn, idx)
    loss, perp = _tc_scalars(maxsim.reshape(128, m // 128), counts, m, d)
    return zq.reshape(b, n, d), loss.reshape(()), perp.reshape(())


# speedup vs baseline: 2.4261x; 1.0157x over previous
import functools

import jax
import jax.numpy as jnp
from jax import lax
from jax.experimental import pallas as pl
from jax.experimental.pallas import tpu as pltpu
from jax.experimental.pallas import tpu_sc as plsc


def _tc_argmax(x_flat, codebook, mt=256):
    m, d = x_flat.shape
    k_total = codebook.shape[0]
    n_mt = m // mt

    def body(x_ref, cb_ref, cn_ref, idx_ref, ms_ref):
        mi = pl.program_id(0)

        @pl.when(mi == 0)
        def _():
            cb = cb_ref[...]
            cn_ref[...] = cb / jnp.maximum(
                jnp.sqrt(jnp.sum(cb * cb, axis=1, keepdims=True)), 1e-12)

        xb = x_ref[...]
        xn = xb / jnp.maximum(
            jnp.sqrt(jnp.sum(xb * xb, axis=1, keepdims=True)), 1e-12)
        sim = lax.dot_general(xn, cn_ref[...], (((1,), (1,)), ((), ())),
                              preferred_element_type=jnp.float32)
        tmax = jnp.max(sim, axis=1)
        targ = jnp.argmax(sim, axis=1).astype(jnp.int32)
        idx_ref[...] = targ
        ms_ref[...] = tmax

    return pl.pallas_call(
        body,
        grid=(n_mt,),
        in_specs=[
            pl.BlockSpec((mt, d), lambda mi: (mi, 0)),
            pl.BlockSpec((k_total, d), lambda mi: (0, 0)),
        ],
        out_specs=[
            pl.BlockSpec((k_total, d), lambda mi: (0, 0)),
            pl.BlockSpec((mt,), lambda mi: (mi,)),
            pl.BlockSpec((mt,), lambda mi: (mi,)),
        ],
        out_shape=[
            jax.ShapeDtypeStruct((k_total, d), jnp.float32),
            jax.ShapeDtypeStruct((m,), jnp.int32),
            jax.ShapeDtypeStruct((m,), jnp.float32),
        ],
        compiler_params=pltpu.CompilerParams(
            vmem_limit_bytes=100 * 1024 * 1024),
    )(x_flat, codebook)


def _sc_gather_hist(cn, idx):
    k_total, d = cn.shape
    m = idx.shape[0]
    info = plsc.get_sparse_core_info()
    nc, ns = info.num_cores, info.num_subcores
    nw = nc * ns
    rows_w = m // nw
    ch = 128
    n_ch = rows_w // ch
    hr, hc = k_total // 128, 128

    mesh = plsc.VectorSubcoreMesh(core_axis_name="c", subcore_axis_name="s")

    @functools.partial(
        pl.kernel,
        out_type=[
            jax.ShapeDtypeStruct((m, d), jnp.float32),
            jax.ShapeDtypeStruct((nc, hr, hc), jnp.float32),
        ],
        mesh=mesh,
        compiler_params=pltpu.CompilerParams(needs_layout_passes=False),
        scratch_types=[
            pltpu.VMEM((rows_w,), jnp.int32),
            pltpu.VMEM((ch, d), jnp.float32),
            pltpu.VMEM((ch, d), jnp.float32),
            pltpu.VMEM((hr, hc), jnp.float32),
            pltpu.VMEM((hr,), jnp.int32),
            pltpu.VMEM_SHARED((hr, hc), jnp.float32),
            pltpu.SemaphoreType.DMA,
            pltpu.SemaphoreType.DMA,
        ],
    )
    def sc_body(cn_hbm, idx_hbm, zq_hbm, cnt_hbm,
                idx_v, buf0, buf1, hist_v, rowid_v, shared_hist, sem0, sem1):
        ci = lax.axis_index("c")
        si = lax.axis_index("s")
        wid = si * nc + ci
        base = wid * rows_w

        pltpu.sync_copy(idx_hbm.at[pl.ds(base, rows_w)], idx_v)

        zeros16 = jnp.zeros((16,), jnp.float32)

        def zero_body(t, _):
            r = t // (hc // 16)
            c = (t % (hc // 16)) * 16
            hist_v[r, pl.ds(c, 16)] = zeros16
            return 0

        lax.fori_loop(0, hr * (hc // 16), zero_body, 0)
        for j in range(hr // 16):
            rowid_v[pl.ds(j * 16, 16)] = lax.iota(jnp.int32, 16) + j * 16

        @pl.when(si == 0)
        def _():
            pltpu.sync_copy(hist_v, shared_hist)

        bufs = (buf0, buf1)
        sems = (sem0, sem1)
        cp = pltpu.async_copy(cn_hbm.at[idx_v.at[pl.ds(0, ch)]], buf0, sem0)
        for c in range(n_ch):
            nxt = None
            if c + 1 < n_ch:
                nxt = pltpu.async_copy(
                    cn_hbm.at[idx_v.at[pl.ds((c + 1) * ch, ch)]],
                    bufs[(c + 1) % 2], sems[(c + 1) % 2])
            cp.wait()
            pltpu.sync_copy(bufs[c % 2], zq_hbm.at[pl.ds(base + c * ch, ch)])
            cp = nxt

        lane = lax.iota(jnp.int32, 16)
        ones16 = jnp.ones((16,), jnp.float32)

        def hist_body(v, _):
            vec = idx_v[pl.ds(v * 16, 16)]
            row = lax.shift_right_logical(vec, 7)
            col = lax.bitwise_and(vec, 127)
            for j in range(16):
                plsc.addupdate_scatter(hist_v, [row, col], ones16,
                                       mask=lane == j)
            return 0

        lax.fori_loop(0, rows_w // 16, hist_body, 0)

        plsc.subcore_barrier()
        pltpu.sync_copy(hist_v, shared_hist.at[rowid_v], add=True)
        plsc.subcore_barrier()

        @pl.when(si == 0)
        def _():
            pltpu.sync_copy(shared_hist, cnt_hbm.at[ci])

    return sc_body(cn, idx)


def _tc_scalars(ms2d, counts, m, d):
    def body(ms_ref, cnt_ref, loss_ref, perp_ref):
        mean_s = jnp.sum(ms_ref[...]) * (1.0 / m)
        loss = 1.25 * (2.0 - 2.0 * mean_s) * (1.0 / d)
        loss_ref[...] = jnp.broadcast_to(loss, (1, 1))
        cnt = cnt_ref[0] + cnt_ref[1]
        e = cnt * (1.0 / m)
        ent = -jnp.sum(e * jnp.log(e + 1e-10))
        perp_ref[...] = jnp.broadcast_to(jnp.exp(ent), (1, 1))

    return pl.pallas_call(
        body,
        out_shape=[
            jax.ShapeDtypeStruct((1, 1), jnp.float32),
            jax.ShapeDtypeStruct((1, 1), jnp.float32),
        ],
    )(ms2d, counts)


def kernel(x, codebook):
    b, n, d = x.shape
    m = b * n
    x_flat = x.reshape(m, d)
    cn, idx, maxsim = _tc_argmax(x_flat, codebook)
    zq, counts = _sc_gather_hist(cn, idx)
    loss, perp = _tc_scalars(maxsim.reshape(128, m // 128), counts, m, d)
    return zq.reshape(b, n, d), loss.reshape(()), perp.reshape(())
